# baseline (speedup 1.0000x reference)
import functools

import jax
import jax.numpy as jnp
from jax import lax
from jax.experimental import pallas as pl
from jax.experimental.pallas import tpu as pltpu
from jax.experimental.pallas import tpu_sc as plsc

N = 10000
NP = 10240
D = 16
E = 320000
NSC = 2
NTILE = 16
NW = NSC * NTILE
CHUNK = 128
NCHUNK = 80
NBUF = 8
NGRP = NCHUNK // NBUF
EPW = CHUNK * NCHUNK
EP = EPW * NW
ROWS_PER_TILE = NP // NTILE

NPP = NP * D // 128
BP = NPP // 8
GRID = 8
EROWS = 2 * E // 128
ECROWS = E // 128
PROWS = EP // 128

_MESH = plsc.VectorSubcoreMesh(core_axis_name="c", subcore_axis_name="s")
_SC_PARAMS = pltpu.CompilerParams(use_tc_tiling_on_sc=False)


def _sc_deg_body(col_hbm, ones_hbm, zero_hbm, out_a, out_b, col_v, ones_v,
                 accum_s):
    c = lax.axis_index("c")
    s = lax.axis_index("s")
    wid = c * NTILE + s
    rbase = s * ROWS_PER_TILE

    for k in range(ROWS_PER_TILE // CHUNK):
        pltpu.sync_copy(zero_hbm, accum_s.at[pl.ds(rbase + k * CHUNK, CHUNK)])
    pltpu.sync_copy(col_hbm.at[wid], col_v)
    pltpu.sync_copy(ones_hbm, ones_v)
    plsc.subcore_barrier()

    def body(j, carry):
        pltpu.sync_copy(ones_v, accum_s.at[col_v.at[j]], add=True)
        return carry

    lax.fori_loop(0, NCHUNK, body, 0)

    plsc.subcore_barrier()

    @pl.when(c == 0)
    def _():
        pltpu.sync_copy(accum_s.at[pl.ds(rbase, ROWS_PER_TILE)],
                        out_a.at[pl.ds(rbase, ROWS_PER_TILE)])

    @pl.when(c != 0)
    def _():
        pltpu.sync_copy(accum_s.at[pl.ds(rbase, ROWS_PER_TILE)],
                        out_b.at[pl.ds(rbase, ROWS_PER_TILE)])


_sc_deg = functools.partial(
    pl.kernel,
    out_type=[jax.ShapeDtypeStruct((NP, D), jnp.float32),
              jax.ShapeDtypeStruct((NP, D), jnp.float32)],
    mesh=_MESH,
    scratch_types=[
        pltpu.VMEM((NCHUNK, CHUNK), jnp.int32),
        pltpu.VMEM((CHUNK, D), jnp.float32),
        pltpu.VMEM_SHARED((NP, D), jnp.float32),
    ],
    compiler_params=_SC_PARAMS,
)(_sc_deg_body)


def _sc_agg_body(g_hbm, row_hbm, col_hbm, zero_hbm, out_a, out_b,
                 row_v, col_v, data_v, accum_s,
                 g0, g1, g2, g3, g4, g5, g6, g7,
                 t0, t1, t2, t3, t4, t5, t6, t7):
    c = lax.axis_index("c")
    s = lax.axis_index("s")
    wid = c * NTILE + s
    rbase = s * ROWS_PER_TILE
    gsems = (g0, g1, g2, g3, g4, g5, g6, g7)
    ssems = (t0, t1, t2, t3, t4, t5, t6, t7)

    @pl.when(c == 0)
    def _():
        pltpu.sync_copy(g_hbm.at[pl.ds(rbase, ROWS_PER_TILE)],
                        accum_s.at[pl.ds(rbase, ROWS_PER_TILE)])

    @pl.when(c != 0)
    def _():
        for k in range(ROWS_PER_TILE // CHUNK):
            pltpu.sync_copy(zero_hbm,
                            accum_s.at[pl.ds(rbase + k * CHUNK, CHUNK)])

    pltpu.sync_copy(row_hbm.at[wid], row_v)
    pltpu.sync_copy(col_hbm.at[wid], col_v)
    plsc.subcore_barrier()

    for b in range(NBUF):
        pltpu.async_copy(g_hbm.at[row_v.at[b]], data_v.at[b], gsems[b])

    def group(gi, carry):
        for b in range(NBUF):
            j = gi * NBUF + b
            pltpu.make_async_copy(g_hbm.at[row_v.at[j]],
                                  data_v.at[b], gsems[b]).wait()
            pltpu.async_copy(data_v.at[b], accum_s.at[col_v.at[j]],
                             ssems[b], add=True)
        for b in range(NBUF):
            j = gi * NBUF + b
            pltpu.make_async_copy(data_v.at[b], accum_s.at[col_v.at[j]],
                                  ssems[b]).wait()

            @pl.when(j + NBUF < NCHUNK)
            def _():
                pltpu.async_copy(g_hbm.at[row_v.at[j + NBUF]],
                                 data_v.at[b], gsems[b])
        return carry

    lax.fori_loop(0, NGRP, group, 0)

    plsc.subcore_barrier()

    @pl.when(c == 0)
    def _():
        pltpu.sync_copy(accum_s.at[pl.ds(rbase, ROWS_PER_TILE)],
                        out_a.at[pl.ds(rbase, ROWS_PER_TILE)])

    @pl.when(c != 0)
    def _():
        pltpu.sync_copy(accum_s.at[pl.ds(rbase, ROWS_PER_TILE)],
                        out_b.at[pl.ds(rbase, ROWS_PER_TILE)])


_sc_agg = functools.partial(
    pl.kernel,
    out_type=[jax.ShapeDtypeStruct((NP, D), jnp.float32),
              jax.ShapeDtypeStruct((NP, D), jnp.float32)],
    mesh=_MESH,
    scratch_types=[
        pltpu.VMEM((NCHUNK, CHUNK), jnp.int32),
        pltpu.VMEM((NCHUNK, CHUNK), jnp.int32),
        pltpu.VMEM((NBUF, CHUNK, D), jnp.float32),
        pltpu.VMEM_SHARED((NP, D), jnp.float32),
    ] + [pltpu.SemaphoreType.DMA] * (2 * NBUF),
    compiler_params=_SC_PARAMS,
)(_sc_agg_body)


def _tc_prep_body(ei_ref, row_ref, col_ref):
    rv = ei_ref[0:ECROWS, :]
    cv = ei_ref[ECROWS:EROWS, :]
    npad = PROWS - ECROWS
    pr = (lax.broadcasted_iota(jnp.int32, (npad, 128), 0) * 128
          + lax.broadcasted_iota(jnp.int32, (npad, 128), 1))
    rowfull = jnp.concatenate([rv, pr % N], axis=0)
    colfull = jnp.concatenate([cv, N + pr % (NP - N)], axis=0)
    row_ref[...] = (rowfull % NPP) * 8 + rowfull // NPP
    col_ref[...] = (colfull % NPP) * 8 + colfull // NPP


def _tc_prep(ei32):
    out = jax.ShapeDtypeStruct((PROWS, 128), jnp.int32)
    full = pl.BlockSpec((PROWS, 128), lambda i: (0, 0))
    return pl.pallas_call(
        _tc_prep_body,
        grid=(1,),
        in_specs=[pl.BlockSpec((EROWS, 128), lambda i: (0, 0))],
        out_specs=[full, full],
        out_shape=[out, out],
    )(ei32)


_pk_spec = pl.BlockSpec((BP, 128), lambda i: (i, 0))
_b_spec = pl.BlockSpec((8, 128), lambda i: (0, 0))


def _tc_mm_body(x_ref, w_ref, h_ref):
    h = jnp.dot(x_ref[...], w_ref[...], preferred_element_type=jnp.float32)
    h_ref[...] = jnp.concatenate(
        [h[NPP * i:NPP * (i + 1)] for i in range(8)], axis=1)


def _tc_mm(x_p, w1):
    return pl.pallas_call(
        _tc_mm_body,
        grid=(1,),
        in_specs=[
            pl.BlockSpec((NP, 128), lambda i: (0, 0)),
            pl.BlockSpec((128, D), lambda i: (0, 0)),
        ],
        out_specs=pl.BlockSpec((NPP, 128), lambda i: (0, 0)),
        out_shape=jax.ShapeDtypeStruct((NPP, 128), jnp.float32),
    )(x_p, w1)


def _tc_scale_body(h_ref, da_ref, db_ref, g_ref, dinv_ref):
    deg = da_ref[...] + db_ref[...] + 1.0
    dinv = lax.rsqrt(deg)
    dinv_ref[...] = dinv
    g_ref[...] = h_ref[...] * dinv


def _tc_scale(h, deg_a, deg_b):
    return pl.pallas_call(
        _tc_scale_body,
        grid=(GRID,),
        in_specs=[_pk_spec, _pk_spec, _pk_spec],
        out_specs=[_pk_spec, _pk_spec],
        out_shape=[jax.ShapeDtypeStruct((NPP, 128), jnp.float32),
                   jax.ShapeDtypeStruct((NPP, 128), jnp.float32)],
    )(h, deg_a, deg_b)


def _tc_mid1_body(aa_ref, ab_ref, dinv_ref, b_ref, g_ref):
    dinv = dinv_ref[...]
    z = (aa_ref[...] + ab_ref[...]) * dinv + b_ref[0:1, :]
    g_ref[...] = jnp.maximum(z, 0.0) * dinv


def _tc_mid1(acc_a, acc_b, dinv, b_pack):
    return pl.pallas_call(
        _tc_mid1_body,
        grid=(GRID,),
        in_specs=[_pk_spec, _pk_spec, _pk_spec, _b_spec],
        out_specs=_pk_spec,
        out_shape=jax.ShapeDtypeStruct((NPP, 128), jnp.float32),
    )(acc_a, acc_b, dinv, b_pack)


def _tc_mid2_body(aa_ref, ab_ref, dinv_ref, b_ref, wk_ref, g_ref):
    dinv = dinv_ref[...]
    t = (aa_ref[...] + ab_ref[...]) * dinv
    z = jnp.dot(t, wk_ref[...], preferred_element_type=jnp.float32)
    g_ref[...] = jnp.maximum(z + b_ref[0:1, :], 0.0) * dinv


def _tc_mid2(acc_a, acc_b, dinv, b_pack, w_kron):
    return pl.pallas_call(
        _tc_mid2_body,
        grid=(GRID,),
        in_specs=[
            _pk_spec,
            _pk_spec,
            _pk_spec,
            _b_spec,
            pl.BlockSpec((128, 128), lambda i: (0, 0)),
        ],
        out_specs=_pk_spec,
        out_shape=jax.ShapeDtypeStruct((NPP, 128), jnp.float32),
    )(acc_a, acc_b, dinv, b_pack, w_kron)


def _tc_fin_body(aa_ref, ab_ref, dinv_ref, w_ref, b_ref, o_ref):
    t = (aa_ref[...] + ab_ref[...]) * dinv_ref[...]
    r = jnp.concatenate([t[:, D * i:D * (i + 1)] for i in range(8)], axis=0)
    z = jnp.dot(r, w_ref[...], preferred_element_type=jnp.float32) + b_ref[0:1, :]
    m = jnp.max(z, axis=1, keepdims=True)
    zs = z - m
    o_ref[...] = zs - jnp.log(jnp.sum(jnp.exp(zs), axis=1, keepdims=True))


def _tc_fin(acc_a, acc_b, dinv, w2, b2_tiled):
    full = pl.BlockSpec((NPP, 128), lambda i: (0, 0))
    return pl.pallas_call(
        _tc_fin_body,
        grid=(1,),
        in_specs=[
            full,
            full,
            full,
            pl.BlockSpec((D, 2), lambda i: (0, 0)),
            pl.BlockSpec((8, 2), lambda i: (0, 0)),
        ],
        out_specs=pl.BlockSpec((NP, 2), lambda i: (0, 0)),
        out_shape=jax.ShapeDtypeStruct((NP, 2), jnp.float32),
    )(acc_a, acc_b, dinv, w2, b2_tiled)


def _packed(a):
    return a.reshape(NPP, 128)


def kernel(x, edge_index, W1, b1, W3, b3, W2, b2):
    ei32 = edge_index.astype(jnp.int32).reshape(EROWS, 128)
    row2, col2 = _tc_prep(ei32)
    row3 = row2.reshape(NW, NCHUNK, CHUNK)
    col3 = col2.reshape(NW, NCHUNK, CHUNK)

    x_p = jnp.pad(x, ((0, NP - N), (0, 0)))
    zeros = jnp.zeros((CHUNK, D), jnp.float32)
    ones_chunk = jnp.ones((CHUNK, D), jnp.float32)
    b1_pk = jnp.tile(b1.reshape(1, D), (8, 8))
    b3_pk = jnp.tile(b3.reshape(1, D), (8, 8))
    b2_t = jnp.tile(b2.reshape(1, 2), (8, 1))
    w3_kron = jnp.kron(jnp.eye(8, dtype=jnp.float32), W3)

    h1 = _tc_mm(x_p, W1)
    deg_a, deg_b = _sc_deg(col3, ones_chunk, zeros)
    g1, dinv = _tc_scale(h1, _packed(deg_a), _packed(deg_b))
    a1, b1_ = _sc_agg(g1.reshape(NP, D), row3, col3, zeros)
    g2 = _tc_mid1(_packed(a1), _packed(b1_), dinv, b1_pk)
    a2, b2_ = _sc_agg(g2.reshape(NP, D), row3, col3, zeros)
    g3 = _tc_mid2(_packed(a2), _packed(b2_), dinv, b3_pk, w3_kron)
    a3, b3_ = _sc_agg(g3.reshape(NP, D), row3, col3, zeros)
    out = _tc_fin(_packed(a3), _packed(b3_), dinv, W2, b2_t)
    return out[:N]


# speedup vs baseline: 1.1008x; 1.1008x over previous
import functools

import jax
import jax.numpy as jnp
from jax import lax
from jax.experimental import pallas as pl
from jax.experimental.pallas import tpu as pltpu
from jax.experimental.pallas import tpu_sc as plsc

N = 10000
NP = 10240
D = 16
E = 320000
NSC = 2
NTILE = 16
NW = NSC * NTILE
CHUNK = 128
NCHUNK = 80
NBUF = 8
NGRP = NCHUNK // NBUF
EPW = CHUNK * NCHUNK
EP = EPW * NW
ROWS_PER_TILE = NP // NTILE

NPP = NP * D // 128
BP = NPP // 8
GRID = 8
EROWS = 2 * E // 128
ECROWS = E // 128
PROWS = EP // 128

_MESH = plsc.VectorSubcoreMesh(core_axis_name="c", subcore_axis_name="s")
_SC_PARAMS = pltpu.CompilerParams(use_tc_tiling_on_sc=False)


def _sc_deg_body(col_hbm, ones_hbm, zero_hbm, out_a, out_b, col_v, ones_v,
                 accum_s):
    c = lax.axis_index("c")
    s = lax.axis_index("s")
    wid = c * NTILE + s
    rbase = s * ROWS_PER_TILE

    pltpu.sync_copy(zero_hbm.at[pl.ds(rbase, ROWS_PER_TILE)],
                    accum_s.at[pl.ds(rbase, ROWS_PER_TILE)])
    pltpu.sync_copy(col_hbm.at[wid], col_v)
    pltpu.sync_copy(ones_hbm, ones_v)
    plsc.subcore_barrier()

    def body(j, carry):
        pltpu.sync_copy(ones_v, accum_s.at[col_v.at[j]], add=True)
        return carry

    lax.fori_loop(0, NCHUNK, body, 0)

    plsc.subcore_barrier()

    @pl.when(c == 0)
    def _():
        pltpu.sync_copy(accum_s.at[pl.ds(rbase, ROWS_PER_TILE)],
                        out_a.at[pl.ds(rbase, ROWS_PER_TILE)])

    @pl.when(c != 0)
    def _():
        pltpu.sync_copy(accum_s.at[pl.ds(rbase, ROWS_PER_TILE)],
                        out_b.at[pl.ds(rbase, ROWS_PER_TILE)])


_sc_deg = functools.partial(
    pl.kernel,
    out_type=[jax.ShapeDtypeStruct((NP, D), jnp.float32),
              jax.ShapeDtypeStruct((NP, D), jnp.float32)],
    mesh=_MESH,
    scratch_types=[
        pltpu.VMEM((NCHUNK, CHUNK), jnp.int32),
        pltpu.VMEM((CHUNK, D), jnp.float32),
        pltpu.VMEM_SHARED((NP, D), jnp.float32),
    ],
    compiler_params=_SC_PARAMS,
)(_sc_deg_body)


def _sc_agg_body(g_hbm, row_hbm, col_hbm, zero_hbm, out_a, out_b,
                 row_v, col_v, data_v, accum_s,
                 g0, g1, g2, g3, g4, g5, g6, g7,
                 t0, t1, t2, t3, t4, t5, t6, t7):
    c = lax.axis_index("c")
    s = lax.axis_index("s")
    wid = c * NTILE + s
    rbase = s * ROWS_PER_TILE
    gsems = (g0, g1, g2, g3, g4, g5, g6, g7)
    ssems = (t0, t1, t2, t3, t4, t5, t6, t7)

    @pl.when(c == 0)
    def _():
        pltpu.sync_copy(g_hbm.at[pl.ds(rbase, ROWS_PER_TILE)],
                        accum_s.at[pl.ds(rbase, ROWS_PER_TILE)])

    @pl.when(c != 0)
    def _():
        pltpu.sync_copy(zero_hbm.at[pl.ds(rbase, ROWS_PER_TILE)],
                        accum_s.at[pl.ds(rbase, ROWS_PER_TILE)])

    pltpu.sync_copy(row_hbm.at[wid], row_v)
    pltpu.sync_copy(col_hbm.at[wid], col_v)
    plsc.subcore_barrier()

    for b in range(NBUF):
        pltpu.async_copy(g_hbm.at[row_v.at[b]], data_v.at[b], gsems[b])

    def group(gi, carry):
        for b in range(NBUF):
            j = gi * NBUF + b
            pltpu.make_async_copy(g_hbm.at[row_v.at[j]],
                                  data_v.at[b], gsems[b]).wait()
            pltpu.async_copy(data_v.at[b], accum_s.at[col_v.at[j]],
                             ssems[b], add=True)
        for b in range(NBUF):
            j = gi * NBUF + b
            pltpu.make_async_copy(data_v.at[b], accum_s.at[col_v.at[j]],
                                  ssems[b]).wait()

            @pl.when(j + NBUF < NCHUNK)
            def _():
                pltpu.async_copy(g_hbm.at[row_v.at[j + NBUF]],
                                 data_v.at[b], gsems[b])
        return carry

    lax.fori_loop(0, NGRP, group, 0)

    plsc.subcore_barrier()

    @pl.when(c == 0)
    def _():
        pltpu.sync_copy(accum_s.at[pl.ds(rbase, ROWS_PER_TILE)],
                        out_a.at[pl.ds(rbase, ROWS_PER_TILE)])

    @pl.when(c != 0)
    def _():
        pltpu.sync_copy(accum_s.at[pl.ds(rbase, ROWS_PER_TILE)],
                        out_b.at[pl.ds(rbase, ROWS_PER_TILE)])


_sc_agg = functools.partial(
    pl.kernel,
    out_type=[jax.ShapeDtypeStruct((NP, D), jnp.float32),
              jax.ShapeDtypeStruct((NP, D), jnp.float32)],
    mesh=_MESH,
    scratch_types=[
        pltpu.VMEM((NCHUNK, CHUNK), jnp.int32),
        pltpu.VMEM((NCHUNK, CHUNK), jnp.int32),
        pltpu.VMEM((NBUF, CHUNK, D), jnp.float32),
        pltpu.VMEM_SHARED((NP, D), jnp.float32),
    ] + [pltpu.SemaphoreType.DMA] * (2 * NBUF),
    compiler_params=_SC_PARAMS,
)(_sc_agg_body)


def _tc_prep_body(ei_ref, row_ref, col_ref):
    rv = ei_ref[0:ECROWS, :]
    cv = ei_ref[ECROWS:EROWS, :]
    npad = PROWS - ECROWS
    pr = (lax.broadcasted_iota(jnp.int32, (npad, 128), 0) * 128
          + lax.broadcasted_iota(jnp.int32, (npad, 128), 1))
    rowfull = jnp.concatenate([rv, pr % N], axis=0)
    colfull = jnp.concatenate([cv, N + pr % (NP - N)], axis=0)
    row_ref[...] = (rowfull % NPP) * 8 + rowfull // NPP
    col_ref[...] = (colfull % NPP) * 8 + colfull // NPP


def _tc_prep(ei32):
    out = jax.ShapeDtypeStruct((PROWS, 128), jnp.int32)
    full = pl.BlockSpec((PROWS, 128), lambda i: (0, 0))
    return pl.pallas_call(
        _tc_prep_body,
        grid=(1,),
        in_specs=[pl.BlockSpec((EROWS, 128), lambda i: (0, 0))],
        out_specs=[full, full],
        out_shape=[out, out],
    )(ei32)


_pk_spec = pl.BlockSpec((BP, 128), lambda i: (i, 0))
_b_spec = pl.BlockSpec((8, 128), lambda i: (0, 0))


def _tc_mm_body(x_ref, w_ref, h_ref):
    h = jnp.dot(x_ref[...], w_ref[...], preferred_element_type=jnp.float32)
    h_ref[...] = jnp.concatenate(
        [h[NPP * i:NPP * (i + 1)] for i in range(8)], axis=1)


def _tc_mm(x_p, w1):
    return pl.pallas_call(
        _tc_mm_body,
        grid=(1,),
        in_specs=[
            pl.BlockSpec((NP, 128), lambda i: (0, 0)),
            pl.BlockSpec((128, D), lambda i: (0, 0)),
        ],
        out_specs=pl.BlockSpec((NPP, 128), lambda i: (0, 0)),
        out_shape=jax.ShapeDtypeStruct((NPP, 128), jnp.float32),
    )(x_p, w1)


def _tc_scale_body(h_ref, da_ref, db_ref, g_ref, dinv_ref):
    deg = da_ref[...] + db_ref[...] + 1.0
    dinv = lax.rsqrt(deg)
    dinv_ref[...] = dinv
    g_ref[...] = h_ref[...] * dinv


def _tc_scale(h, deg_a, deg_b):
    return pl.pallas_call(
        _tc_scale_body,
        grid=(GRID,),
        in_specs=[_pk_spec, _pk_spec, _pk_spec],
        out_specs=[_pk_spec, _pk_spec],
        out_shape=[jax.ShapeDtypeStruct((NPP, 128), jnp.float32),
                   jax.ShapeDtypeStruct((NPP, 128), jnp.float32)],
    )(h, deg_a, deg_b)


def _tc_mid1_body(aa_ref, ab_ref, dinv_ref, b_ref, g_ref):
    dinv = dinv_ref[...]
    z = (aa_ref[...] + ab_ref[...]) * dinv + b_ref[0:1, :]
    g_ref[...] = jnp.maximum(z, 0.0) * dinv


def _tc_mid1(acc_a, acc_b, dinv, b_pack):
    return pl.pallas_call(
        _tc_mid1_body,
        grid=(GRID,),
        in_specs=[_pk_spec, _pk_spec, _pk_spec, _b_spec],
        out_specs=_pk_spec,
        out_shape=jax.ShapeDtypeStruct((NPP, 128), jnp.float32),
    )(acc_a, acc_b, dinv, b_pack)


def _tc_mid2_body(aa_ref, ab_ref, dinv_ref, b_ref, wk_ref, g_ref):
    dinv = dinv_ref[...]
    t = (aa_ref[...] + ab_ref[...]) * dinv
    z = jnp.dot(t, wk_ref[...], preferred_element_type=jnp.float32)
    g_ref[...] = jnp.maximum(z + b_ref[0:1, :], 0.0) * dinv


def _tc_mid2(acc_a, acc_b, dinv, b_pack, w_kron):
    return pl.pallas_call(
        _tc_mid2_body,
        grid=(GRID,),
        in_specs=[
            _pk_spec,
            _pk_spec,
            _pk_spec,
            _b_spec,
            pl.BlockSpec((128, 128), lambda i: (0, 0)),
        ],
        out_specs=_pk_spec,
        out_shape=jax.ShapeDtypeStruct((NPP, 128), jnp.float32),
    )(acc_a, acc_b, dinv, b_pack, w_kron)


def _tc_fin_body(aa_ref, ab_ref, dinv_ref, w_ref, b_ref, o_ref):
    t = (aa_ref[...] + ab_ref[...]) * dinv_ref[...]
    r = jnp.concatenate([t[:, D * i:D * (i + 1)] for i in range(8)], axis=0)
    z = jnp.dot(r, w_ref[...], preferred_element_type=jnp.float32) + b_ref[0:1, :]
    m = jnp.max(z, axis=1, keepdims=True)
    zs = z - m
    o_ref[...] = zs - jnp.log(jnp.sum(jnp.exp(zs), axis=1, keepdims=True))


def _tc_fin(acc_a, acc_b, dinv, w2, b2_tiled):
    full = pl.BlockSpec((NPP, 128), lambda i: (0, 0))
    return pl.pallas_call(
        _tc_fin_body,
        grid=(1,),
        in_specs=[
            full,
            full,
            full,
            pl.BlockSpec((D, 2), lambda i: (0, 0)),
            pl.BlockSpec((8, 2), lambda i: (0, 0)),
        ],
        out_specs=pl.BlockSpec((NP, 2), lambda i: (0, 0)),
        out_shape=jax.ShapeDtypeStruct((NP, 2), jnp.float32),
    )(acc_a, acc_b, dinv, w2, b2_tiled)


def _packed(a):
    return a.reshape(NPP, 128)


def kernel(x, edge_index, W1, b1, W3, b3, W2, b2):
    ei32 = edge_index.astype(jnp.int32).reshape(EROWS, 128)
    row2, col2 = _tc_prep(ei32)
    row3 = row2.reshape(NW, NCHUNK, CHUNK)
    col3 = col2.reshape(NW, NCHUNK, CHUNK)

    x_p = jnp.pad(x, ((0, NP - N), (0, 0)))
    zeros = jnp.zeros((NP, D), jnp.float32)
    ones_chunk = jnp.ones((CHUNK, D), jnp.float32)
    b1_pk = jnp.tile(b1.reshape(1, D), (8, 8))
    b3_pk = jnp.tile(b3.reshape(1, D), (8, 8))
    b2_t = jnp.tile(b2.reshape(1, 2), (8, 1))
    w3_kron = jnp.kron(jnp.eye(8, dtype=jnp.float32), W3)

    h1 = _tc_mm(x_p, W1)
    deg_a, deg_b = _sc_deg(col3, ones_chunk, zeros)
    g1, dinv = _tc_scale(h1, _packed(deg_a), _packed(deg_b))
    a1, b1_ = _sc_agg(g1.reshape(NP, D), row3, col3, zeros)
    g2 = _tc_mid1(_packed(a1), _packed(b1_), dinv, b1_pk)
    a2, b2_ = _sc_agg(g2.reshape(NP, D), row3, col3, zeros)
    g3 = _tc_mid2(_packed(a2), _packed(b2_), dinv, b3_pk, w3_kron)
    a3, b3_ = _sc_agg(g3.reshape(NP, D), row3, col3, zeros)
    out = _tc_fin(_packed(a3), _packed(b3_), dinv, W2, b2_t)
    return out[:N]


# speedup vs baseline: 1.1068x; 1.0055x over previous
import functools

import jax
import jax.numpy as jnp
from jax import lax
from jax.experimental import pallas as pl
from jax.experimental.pallas import tpu as pltpu
from jax.experimental.pallas import tpu_sc as plsc

N = 10000
NP = 10240
D = 16
E = 320000
NSC = 2
NTILE = 16
NW = NSC * NTILE
CHUNK = 128
NCHUNK = 80
NBUF = 8
NGRP = NCHUNK // NBUF
EPW = CHUNK * NCHUNK
EP = EPW * NW
ROWS_PER_TILE = NP // NTILE

NPP = NP * D // 128
BP = NPP // 8
GRID = 8
EROWS = 2 * E // 128
ECROWS = E // 128
PROWS = EP // 128

_MESH = plsc.VectorSubcoreMesh(core_axis_name="c", subcore_axis_name="s")
_SC_PARAMS = pltpu.CompilerParams(use_tc_tiling_on_sc=False)


def _sc_deg_body(col_hbm, ones_hbm, zero_hbm, out_a, out_b, col_v, ones_v,
                 accum_s):
    c = lax.axis_index("c")
    s = lax.axis_index("s")
    wid = c * NTILE + s
    rbase = s * ROWS_PER_TILE

    pltpu.sync_copy(zero_hbm.at[pl.ds(rbase, ROWS_PER_TILE)],
                    accum_s.at[pl.ds(rbase, ROWS_PER_TILE)])
    pltpu.sync_copy(col_hbm.at[wid], col_v)
    pltpu.sync_copy(ones_hbm, ones_v)
    plsc.subcore_barrier()

    def body(j, carry):
        pltpu.sync_copy(ones_v, accum_s.at[col_v.at[j]], add=True)
        return carry

    lax.fori_loop(0, NCHUNK, body, 0)

    plsc.subcore_barrier()

    @pl.when(c == 0)
    def _():
        pltpu.sync_copy(accum_s.at[pl.ds(rbase, ROWS_PER_TILE)],
                        out_a.at[pl.ds(rbase, ROWS_PER_TILE)])

    @pl.when(c != 0)
    def _():
        pltpu.sync_copy(accum_s.at[pl.ds(rbase, ROWS_PER_TILE)],
                        out_b.at[pl.ds(rbase, ROWS_PER_TILE)])


_sc_deg = functools.partial(
    pl.kernel,
    out_type=[jax.ShapeDtypeStruct((NP, D), jnp.float32),
              jax.ShapeDtypeStruct((NP, D), jnp.float32)],
    mesh=_MESH,
    scratch_types=[
        pltpu.VMEM((NCHUNK, CHUNK), jnp.int32),
        pltpu.VMEM((CHUNK, D), jnp.float32),
        pltpu.VMEM_SHARED((NP, D), jnp.float32),
    ],
    compiler_params=_SC_PARAMS,
)(_sc_deg_body)


def _sc_agg_body(g_hbm, row_hbm, col_hbm, zero_hbm, out_a, out_b,
                 row_v, col_v, data_v, accum_s,
                 g0, g1, g2, g3, g4, g5, g6, g7,
                 t0, t1, t2, t3, t4, t5, t6, t7):
    c = lax.axis_index("c")
    s = lax.axis_index("s")
    wid = c * NTILE + s
    rbase = s * ROWS_PER_TILE
    gsems = (g0, g1, g2, g3, g4, g5, g6, g7)
    ssems = (t0, t1, t2, t3, t4, t5, t6, t7)

    pltpu.sync_copy(zero_hbm.at[pl.ds(rbase, ROWS_PER_TILE)],
                    accum_s.at[pl.ds(rbase, ROWS_PER_TILE)])

    pltpu.sync_copy(row_hbm.at[wid], row_v)
    pltpu.sync_copy(col_hbm.at[wid], col_v)
    plsc.subcore_barrier()

    for b in range(NBUF):
        pltpu.async_copy(g_hbm.at[row_v.at[b]], data_v.at[b], gsems[b])

    def group(gi, carry):
        for b in range(NBUF):
            j = gi * NBUF + b
            pltpu.make_async_copy(g_hbm.at[row_v.at[j]],
                                  data_v.at[b], gsems[b]).wait()
            pltpu.async_copy(data_v.at[b], accum_s.at[col_v.at[j]],
                             ssems[b], add=True)
        for b in range(NBUF):
            j = gi * NBUF + b
            pltpu.make_async_copy(data_v.at[b], accum_s.at[col_v.at[j]],
                                  ssems[b]).wait()

            @pl.when(j + NBUF < NCHUNK)
            def _():
                pltpu.async_copy(g_hbm.at[row_v.at[j + NBUF]],
                                 data_v.at[b], gsems[b])
        return carry

    lax.fori_loop(0, NGRP, group, 0)

    plsc.subcore_barrier()

    @pl.when(c == 0)
    def _():
        pltpu.sync_copy(accum_s.at[pl.ds(rbase, ROWS_PER_TILE)],
                        out_a.at[pl.ds(rbase, ROWS_PER_TILE)])

    @pl.when(c != 0)
    def _():
        pltpu.sync_copy(accum_s.at[pl.ds(rbase, ROWS_PER_TILE)],
                        out_b.at[pl.ds(rbase, ROWS_PER_TILE)])


_sc_agg = functools.partial(
    pl.kernel,
    out_type=[jax.ShapeDtypeStruct((NP, D), jnp.float32),
              jax.ShapeDtypeStruct((NP, D), jnp.float32)],
    mesh=_MESH,
    scratch_types=[
        pltpu.VMEM((NCHUNK, CHUNK), jnp.int32),
        pltpu.VMEM((NCHUNK, CHUNK), jnp.int32),
        pltpu.VMEM((NBUF, CHUNK, D), jnp.float32),
        pltpu.VMEM_SHARED((NP, D), jnp.float32),
    ] + [pltpu.SemaphoreType.DMA] * (2 * NBUF),
    compiler_params=_SC_PARAMS,
)(_sc_agg_body)


def _tc_prep_body(ei_ref, row_ref, col_ref):
    rv = ei_ref[0:ECROWS, :]
    cv = ei_ref[ECROWS:EROWS, :]
    npad = PROWS - ECROWS
    pr = (lax.broadcasted_iota(jnp.int32, (npad, 128), 0) * 128
          + lax.broadcasted_iota(jnp.int32, (npad, 128), 1))
    rowfull = jnp.concatenate([rv, pr % N], axis=0)
    colfull = jnp.concatenate([cv, N + pr % (NP - N)], axis=0)
    row_ref[...] = (rowfull % NPP) * 8 + rowfull // NPP
    col_ref[...] = (colfull % NPP) * 8 + colfull // NPP


def _tc_prep(ei32):
    out = jax.ShapeDtypeStruct((PROWS, 128), jnp.int32)
    full = pl.BlockSpec((PROWS, 128), lambda i: (0, 0))
    return pl.pallas_call(
        _tc_prep_body,
        grid=(1,),
        in_specs=[pl.BlockSpec((EROWS, 128), lambda i: (0, 0))],
        out_specs=[full, full],
        out_shape=[out, out],
    )(ei32)


_pk_spec = pl.BlockSpec((BP, 128), lambda i: (i, 0))
_b_spec = pl.BlockSpec((8, 128), lambda i: (0, 0))


def _tc_mm_body(x_ref, w_ref, h_ref):
    h = jnp.dot(x_ref[...], w_ref[...], preferred_element_type=jnp.float32)
    h_ref[...] = jnp.concatenate(
        [h[NPP * i:NPP * (i + 1)] for i in range(8)], axis=1)


def _tc_mm(x_p, w1):
    return pl.pallas_call(
        _tc_mm_body,
        grid=(1,),
        in_specs=[
            pl.BlockSpec((NP, 128), lambda i: (0, 0)),
            pl.BlockSpec((128, D), lambda i: (0, 0)),
        ],
        out_specs=pl.BlockSpec((NPP, 128), lambda i: (0, 0)),
        out_shape=jax.ShapeDtypeStruct((NPP, 128), jnp.float32),
    )(x_p, w1)


def _tc_scale_body(h_ref, da_ref, db_ref, g_ref, dinv_ref):
    deg = da_ref[...] + db_ref[...] + 1.0
    dinv = lax.rsqrt(deg)
    dinv_ref[...] = dinv
    g_ref[...] = h_ref[...] * dinv


def _tc_scale(h, deg_a, deg_b):
    return pl.pallas_call(
        _tc_scale_body,
        grid=(GRID,),
        in_specs=[_pk_spec, _pk_spec, _pk_spec],
        out_specs=[_pk_spec, _pk_spec],
        out_shape=[jax.ShapeDtypeStruct((NPP, 128), jnp.float32),
                   jax.ShapeDtypeStruct((NPP, 128), jnp.float32)],
    )(h, deg_a, deg_b)


def _tc_mid1_body(aa_ref, ab_ref, g_in_ref, dinv_ref, b_ref, g_ref):
    dinv = dinv_ref[...]
    z = (aa_ref[...] + ab_ref[...] + g_in_ref[...]) * dinv + b_ref[0:1, :]
    g_ref[...] = jnp.maximum(z, 0.0) * dinv


def _tc_mid1(acc_a, acc_b, g_in, dinv, b_pack):
    return pl.pallas_call(
        _tc_mid1_body,
        grid=(GRID,),
        in_specs=[_pk_spec, _pk_spec, _pk_spec, _pk_spec, _b_spec],
        out_specs=_pk_spec,
        out_shape=jax.ShapeDtypeStruct((NPP, 128), jnp.float32),
    )(acc_a, acc_b, g_in, dinv, b_pack)


def _tc_mid2_body(aa_ref, ab_ref, g_in_ref, dinv_ref, b_ref, wk_ref, g_ref):
    dinv = dinv_ref[...]
    t = (aa_ref[...] + ab_ref[...] + g_in_ref[...]) * dinv
    z = jnp.dot(t, wk_ref[...], preferred_element_type=jnp.float32)
    g_ref[...] = jnp.maximum(z + b_ref[0:1, :], 0.0) * dinv


def _tc_mid2(acc_a, acc_b, g_in, dinv, b_pack, w_kron):
    return pl.pallas_call(
        _tc_mid2_body,
        grid=(GRID,),
        in_specs=[
            _pk_spec,
            _pk_spec,
            _pk_spec,
            _pk_spec,
            _b_spec,
            pl.BlockSpec((128, 128), lambda i: (0, 0)),
        ],
        out_specs=_pk_spec,
        out_shape=jax.ShapeDtypeStruct((NPP, 128), jnp.float32),
    )(acc_a, acc_b, g_in, dinv, b_pack, w_kron)


def _tc_fin_body(aa_ref, ab_ref, g_in_ref, dinv_ref, w_ref, b_ref, o_ref):
    t = (aa_ref[...] + ab_ref[...] + g_in_ref[...]) * dinv_ref[...]
    r = jnp.concatenate([t[:, D * i:D * (i + 1)] for i in range(8)], axis=0)
    z = jnp.dot(r, w_ref[...], preferred_element_type=jnp.float32) + b_ref[0:1, :]
    m = jnp.max(z, axis=1, keepdims=True)
    zs = z - m
    o_ref[...] = zs - jnp.log(jnp.sum(jnp.exp(zs), axis=1, keepdims=True))


def _tc_fin(acc_a, acc_b, g_in, dinv, w2, b2_tiled):
    full = pl.BlockSpec((NPP, 128), lambda i: (0, 0))
    return pl.pallas_call(
        _tc_fin_body,
        grid=(1,),
        in_specs=[
            full,
            full,
            full,
            full,
            pl.BlockSpec((D, 2), lambda i: (0, 0)),
            pl.BlockSpec((8, 2), lambda i: (0, 0)),
        ],
        out_specs=pl.BlockSpec((NP, 2), lambda i: (0, 0)),
        out_shape=jax.ShapeDtypeStruct((NP, 2), jnp.float32),
    )(acc_a, acc_b, g_in, dinv, w2, b2_tiled)


def _packed(a):
    return a.reshape(NPP, 128)


def kernel(x, edge_index, W1, b1, W3, b3, W2, b2):
    ei32 = edge_index.astype(jnp.int32).reshape(EROWS, 128)
    row2, col2 = _tc_prep(ei32)
    row3 = row2.reshape(NW, NCHUNK, CHUNK)
    col3 = col2.reshape(NW, NCHUNK, CHUNK)

    x_p = jnp.pad(x, ((0, NP - N), (0, 0)))
    zeros = jnp.zeros((NP, D), jnp.float32)
    ones_chunk = jnp.ones((CHUNK, D), jnp.float32)
    b1_pk = jnp.tile(b1.reshape(1, D), (8, 8))
    b3_pk = jnp.tile(b3.reshape(1, D), (8, 8))
    b2_t = jnp.tile(b2.reshape(1, 2), (8, 1))
    w3_kron = jnp.kron(jnp.eye(8, dtype=jnp.float32), W3)

    h1 = _tc_mm(x_p, W1)
    deg_a, deg_b = _sc_deg(col3, ones_chunk, zeros)
    g1, dinv = _tc_scale(h1, _packed(deg_a), _packed(deg_b))
    a1, b1_ = _sc_agg(g1.reshape(NP, D), row3, col3, zeros)
    g2 = _tc_mid1(_packed(a1), _packed(b1_), g1, dinv, b1_pk)
    a2, b2_ = _sc_agg(g2.reshape(NP, D), row3, col3, zeros)
    g3 = _tc_mid2(_packed(a2), _packed(b2_), g2, dinv, b3_pk, w3_kron)
    a3, b3_ = _sc_agg(g3.reshape(NP, D), row3, col3, zeros)
    out = _tc_fin(_packed(a3), _packed(b3_), g3, dinv, W2, b2_t)
    return out[:N]
